# Initial kernel scaffold; baseline (speedup 1.0000x reference)
import jax
import jax.numpy as jnp
from jax.experimental import pallas as pl


def kernel(tokens, table):
    raise NotImplementedError("write your pallas kernel here")


# speedup vs baseline: 9.2384x; 9.2384x over previous
import functools

import jax
import jax.numpy as jnp
from jax import lax
from jax.experimental import pallas as pl
from jax.experimental.pallas import tpu as pltpu
from jax.experimental.pallas import tpu_sc as plsc

WINDOW = 10
BATCH = 1024
SEQ = 200
DIM = 128
OUT = SEQ * DIM // WINDOW
LANES = 16

NUM_CORES = 2
NUM_SUBCORES = 16
NUM_WORKERS = NUM_CORES * NUM_SUBCORES
SENT_PER_TILE = BATCH // NUM_WORKERS
HALF = SEQ // 2

_mesh = plsc.VectorSubcoreMesh(core_axis_name="c", subcore_axis_name="s")


def _gather_sent(table_hbm, tok_v, rows_ref, sem, s):
  c0 = pltpu.make_async_copy(
      table_hbm.at[tok_v.at[2 * s]], rows_ref.at[pl.ds(0, HALF)], sem)
  c1 = pltpu.make_async_copy(
      table_hbm.at[tok_v.at[2 * s + 1]], rows_ref.at[pl.ds(HALF, HALF)], sem)
  return c0, c1


def _pool(rows_ref, out_ref):
  tenl = WINDOW * lax.broadcasted_iota(jnp.int32, (LANES,), 0)

  def body(t, carry):
    base = t * (LANES * WINDOW)
    acc = jnp.zeros((LANES,), jnp.float32)
    for j in range(WINDOW):
      f = tenl + (base + j)
      r = lax.shift_right_logical(f, 7)
      c = lax.bitwise_and(f, 127)
      acc = acc + plsc.load_gather(rows_ref, [r, c])
    out_ref[pl.ds(t * LANES, LANES)] = acc * jnp.float32(1.0 / WINDOW)
    return carry

  lax.fori_loop(0, OUT // LANES, body, 0)


@functools.partial(
    pl.kernel,
    out_type=jax.ShapeDtypeStruct((BATCH, OUT), jnp.float32),
    mesh=_mesh,
    compiler_params=pltpu.CompilerParams(needs_layout_passes=False),
    scratch_types=[
        pltpu.VMEM((2 * SENT_PER_TILE, HALF), jnp.int32),
        pltpu.VMEM((SEQ, DIM), jnp.float32),
        pltpu.VMEM((SEQ, DIM), jnp.float32),
        pltpu.VMEM((OUT,), jnp.float32),
        pltpu.SemaphoreType.DMA,
        pltpu.SemaphoreType.DMA,
    ],
)
def _sc_encode(tok_hbm, table_hbm, out_hbm, tok_v, rows0, rows1, out_v,
               sem0, sem1):
  wid = lax.axis_index("s") * NUM_CORES + lax.axis_index("c")
  sent_base = wid * SENT_PER_TILE

  pltpu.sync_copy(tok_hbm.at[pl.ds(wid * 2 * SENT_PER_TILE, 2 * SENT_PER_TILE)],
                  tok_v)

  bufs = (rows0, rows1)
  sems = (sem0, sem1)

  for c in _gather_sent(table_hbm, tok_v, bufs[0], sems[0], 0):
    c.start()

  def step(s2, carry):
    for b in range(2):
      s = 2 * s2 + b
      for c in _gather_sent(table_hbm, tok_v, bufs[b], sems[b], s):
        c.wait()
      @pl.when(s + 1 < SENT_PER_TILE)
      def _():
        for c in _gather_sent(table_hbm, tok_v, bufs[1 - b], sems[1 - b],
                              s + 1):
          c.start()
      _pool(bufs[b], out_v)
      pltpu.sync_copy(out_v, out_hbm.at[sent_base + s])
    return carry

  lax.fori_loop(0, SENT_PER_TILE // 2, step, 0)


def kernel(tokens, table):
  tok2 = tokens.astype(jnp.int32).reshape(2 * BATCH, HALF)
  return _sc_encode(tok2, table)


# speedup vs baseline: 9.7066x; 1.0507x over previous
import functools

import jax
import jax.numpy as jnp
from jax import lax
from jax.experimental import pallas as pl
from jax.experimental.pallas import tpu as pltpu
from jax.experimental.pallas import tpu_sc as plsc

WINDOW = 10
BATCH = 1024
SEQ = 200
DIM = 128
OUT = SEQ * DIM // WINDOW
LANES = 16

NUM_CORES = 2
NUM_SUBCORES = 16
NUM_WORKERS = NUM_CORES * NUM_SUBCORES
SENT_PER_TILE = BATCH // NUM_WORKERS
HALF = SEQ // 2

_mesh = plsc.VectorSubcoreMesh(core_axis_name="c", subcore_axis_name="s")


def _gather_sent(table_hbm, tok_v, rows_ref, sem, s):
  c0 = pltpu.make_async_copy(
      table_hbm.at[tok_v.at[2 * s]], rows_ref.at[pl.ds(0, HALF)], sem)
  c1 = pltpu.make_async_copy(
      table_hbm.at[tok_v.at[2 * s + 1]], rows_ref.at[pl.ds(HALF, HALF)], sem)
  return c0, c1


UNROLL = 4


def _pool(rows_ref, out_ref):
  tenl = WINDOW * lax.broadcasted_iota(jnp.int32, (LANES,), 0)
  zero = jnp.zeros((LANES,), jnp.int32)

  def body(u, carry):
    base0 = u * (UNROLL * LANES * WINDOW)
    for v in range(UNROLL):
      acc = jnp.zeros((LANES,), jnp.float32)
      for j in range(WINDOW):
        f = tenl + (base0 + (v * LANES * WINDOW + j))
        acc = acc + plsc.load_gather(rows_ref, [zero, f])
      out_ref[pl.ds(u * (UNROLL * LANES) + v * LANES, LANES)] = (
          acc * jnp.float32(1.0 / WINDOW))
    return carry

  lax.fori_loop(0, OUT // (UNROLL * LANES), body, 0)


@functools.partial(
    pl.kernel,
    out_type=jax.ShapeDtypeStruct((BATCH, OUT), jnp.float32),
    mesh=_mesh,
    compiler_params=pltpu.CompilerParams(needs_layout_passes=False),
    scratch_types=[
        pltpu.VMEM((2 * SENT_PER_TILE, HALF), jnp.int32),
        pltpu.VMEM((SEQ, DIM), jnp.float32),
        pltpu.VMEM((SEQ, DIM), jnp.float32),
        pltpu.VMEM((OUT,), jnp.float32),
        pltpu.VMEM((OUT,), jnp.float32),
        pltpu.SemaphoreType.DMA,
        pltpu.SemaphoreType.DMA,
        pltpu.SemaphoreType.DMA,
        pltpu.SemaphoreType.DMA,
    ],
)
def _sc_encode(tok_hbm, table_hbm, out_hbm, tok_v, rows0, rows1, out0, out1,
               sem0, sem1, osem0, osem1):
  wid = lax.axis_index("s") * NUM_CORES + lax.axis_index("c")
  sent_base = wid * SENT_PER_TILE

  pltpu.sync_copy(tok_hbm.at[pl.ds(wid * 2 * SENT_PER_TILE, 2 * SENT_PER_TILE)],
                  tok_v)

  bufs = (rows0, rows1)
  sems = (sem0, sem1)
  outs = (out0, out1)
  osems = (osem0, osem1)

  for c in _gather_sent(table_hbm, tok_v, bufs[0], sems[0], 0):
    c.start()

  def step(s2, carry):
    for b in range(2):
      s = 2 * s2 + b
      for c in _gather_sent(table_hbm, tok_v, bufs[b], sems[b], s):
        c.wait()
      @pl.when(s + 1 < SENT_PER_TILE)
      def _():
        for c in _gather_sent(table_hbm, tok_v, bufs[1 - b], sems[1 - b],
                              s + 1):
          c.start()
      @pl.when(s2 >= 1)
      def _():
        pltpu.make_async_copy(
            outs[b], out_hbm.at[sent_base + s - 2], osems[b]).wait()
      _pool(bufs[b], outs[b])
      pltpu.make_async_copy(outs[b], out_hbm.at[sent_base + s],
                            osems[b]).start()
    return carry

  lax.fori_loop(0, SENT_PER_TILE // 2, step, 0)
  for b in range(2):
    pltpu.make_async_copy(
        outs[b], out_hbm.at[sent_base + SENT_PER_TILE - 2 + b],
        osems[b]).wait()


def kernel(tokens, table):
  tok2 = tokens.astype(jnp.int32).reshape(2 * BATCH, HALF)
  return _sc_encode(tok2, table)


# speedup vs baseline: 13.1888x; 1.3587x over previous
import functools

import jax
import jax.numpy as jnp
from jax import lax
from jax.experimental import pallas as pl
from jax.experimental.pallas import tpu as pltpu
from jax.experimental.pallas import tpu_sc as plsc

WINDOW = 10
BATCH = 1024
SEQ = 200
DIM = 128
OUT = SEQ * DIM // WINDOW
LANES = 16

NUM_CORES = 2
NUM_SUBCORES = 16
NUM_WORKERS = NUM_CORES * NUM_SUBCORES
SENT_PER_TILE = BATCH // NUM_WORKERS
HALF = SEQ // 2

_mesh = plsc.VectorSubcoreMesh(core_axis_name="c", subcore_axis_name="s")


def _gather_sent(table_hbm, tok_v, rows_ref, sem, s):
  c0 = pltpu.make_async_copy(
      table_hbm.at[tok_v.at[2 * s]], rows_ref.at[pl.ds(0, HALF)], sem)
  c1 = pltpu.make_async_copy(
      table_hbm.at[tok_v.at[2 * s + 1]], rows_ref.at[pl.ds(HALF, HALF)], sem)
  return c0, c1


UNROLL = 4


def _pool(rows_ref, out_ref):
  tenl = WINDOW * lax.broadcasted_iota(jnp.int32, (LANES,), 0)
  zero = jnp.zeros((LANES,), jnp.int32)

  @plsc.parallel_loop(0, OUT // LANES, unroll=UNROLL)
  def body(t):
    base = t * (LANES * WINDOW)

    def ld(j):
      return plsc.load_gather(rows_ref, [zero, tenl + (base + j)])

    a = (ld(0) + ld(1)) + (ld(2) + ld(3))
    b = (ld(4) + ld(5)) + (ld(6) + ld(7))
    c = ld(8) + ld(9)
    out_ref[pl.ds(t * LANES, LANES)] = (a + (b + c)) * jnp.float32(
        1.0 / WINDOW)


@functools.partial(
    pl.kernel,
    out_type=jax.ShapeDtypeStruct((BATCH, OUT), jnp.float32),
    mesh=_mesh,
    compiler_params=pltpu.CompilerParams(needs_layout_passes=False),
    scratch_types=[
        pltpu.VMEM((2 * SENT_PER_TILE, HALF), jnp.int32),
        pltpu.VMEM((SEQ, DIM), jnp.float32),
        pltpu.VMEM((SEQ, DIM), jnp.float32),
        pltpu.VMEM((OUT,), jnp.float32),
        pltpu.VMEM((OUT,), jnp.float32),
        pltpu.SemaphoreType.DMA,
        pltpu.SemaphoreType.DMA,
        pltpu.SemaphoreType.DMA,
        pltpu.SemaphoreType.DMA,
    ],
)
def _sc_encode(tok_hbm, table_hbm, out_hbm, tok_v, rows0, rows1, out0, out1,
               sem0, sem1, osem0, osem1):
  wid = lax.axis_index("s") * NUM_CORES + lax.axis_index("c")
  sent_base = wid * SENT_PER_TILE

  pltpu.sync_copy(tok_hbm.at[pl.ds(wid * 2 * SENT_PER_TILE, 2 * SENT_PER_TILE)],
                  tok_v)

  bufs = (rows0, rows1)
  sems = (sem0, sem1)
  outs = (out0, out1)
  osems = (osem0, osem1)

  for c in _gather_sent(table_hbm, tok_v, bufs[0], sems[0], 0):
    c.start()

  def step(s2, carry):
    for b in range(2):
      s = 2 * s2 + b
      for c in _gather_sent(table_hbm, tok_v, bufs[b], sems[b], s):
        c.wait()
      @pl.when(s + 1 < SENT_PER_TILE)
      def _():
        for c in _gather_sent(table_hbm, tok_v, bufs[1 - b], sems[1 - b],
                              s + 1):
          c.start()
      @pl.when(s2 >= 1)
      def _():
        pltpu.make_async_copy(
            outs[b], out_hbm.at[sent_base + s - 2], osems[b]).wait()
      _pool(bufs[b], outs[b])
      pltpu.make_async_copy(outs[b], out_hbm.at[sent_base + s],
                            osems[b]).start()
    return carry

  lax.fori_loop(0, SENT_PER_TILE // 2, step, 0)
  for b in range(2):
    pltpu.make_async_copy(
        outs[b], out_hbm.at[sent_base + SENT_PER_TILE - 2 + b],
        osems[b]).wait()


def kernel(tokens, table):
  tok2 = tokens.astype(jnp.int32).reshape(2 * BATCH, HALF)
  return _sc_encode(tok2, table)


# speedup vs baseline: 16.5081x; 1.2517x over previous
import functools

import jax
import jax.numpy as jnp
from jax import lax
from jax.experimental import pallas as pl
from jax.experimental.pallas import tpu as pltpu
from jax.experimental.pallas import tpu_sc as plsc

WINDOW = 10
BATCH = 1024
SEQ = 200
DIM = 128
OUT = SEQ * DIM // WINDOW
LANES = 16

NUM_CORES = 2
NUM_SUBCORES = 16
NUM_WORKERS = NUM_CORES * NUM_SUBCORES
SENT_PER_TILE = BATCH // NUM_WORKERS
HALF = SEQ // 2

_mesh = plsc.VectorSubcoreMesh(core_axis_name="c", subcore_axis_name="s")


def _gather_sent(table_hbm, tok_v, rows_ref, sem, s):
  c0 = pltpu.make_async_copy(
      table_hbm.at[tok_v.at[2 * s]], rows_ref.at[pl.ds(0, HALF)], sem)
  c1 = pltpu.make_async_copy(
      table_hbm.at[tok_v.at[2 * s + 1]], rows_ref.at[pl.ds(HALF, HALF)], sem)
  return c0, c1


UNROLL = 4


def _pool(rows_ref, out_ref):
  tenl = WINDOW * lax.broadcasted_iota(jnp.int32, (LANES,), 0)
  zero = jnp.zeros((LANES,), jnp.int32)

  @plsc.parallel_loop(0, OUT // LANES, unroll=UNROLL)
  def body(t):
    base = t * (LANES * WINDOW)

    def ld(j):
      return plsc.load_gather(rows_ref, [zero, tenl + (base + j)])

    a = (ld(0) + ld(1)) + (ld(2) + ld(3))
    b = (ld(4) + ld(5)) + (ld(6) + ld(7))
    c = ld(8) + ld(9)
    out_ref[pl.ds(t * LANES, LANES)] = (a + (b + c)) * jnp.float32(
        1.0 / WINDOW)


@functools.partial(
    pl.kernel,
    out_type=jax.ShapeDtypeStruct((BATCH, OUT), jnp.float32),
    mesh=_mesh,
    compiler_params=pltpu.CompilerParams(needs_layout_passes=False),
    scratch_types=[
        pltpu.VMEM((2 * SENT_PER_TILE, HALF), jnp.int32),
        pltpu.VMEM((SEQ, DIM), jnp.float32),
        pltpu.VMEM((SEQ, DIM), jnp.float32),
        pltpu.VMEM((SEQ, DIM), jnp.float32),
        pltpu.VMEM((SEQ, DIM), jnp.float32),
        pltpu.VMEM((OUT,), jnp.float32),
        pltpu.VMEM((OUT,), jnp.float32),
        pltpu.SemaphoreType.DMA,
        pltpu.SemaphoreType.DMA,
        pltpu.SemaphoreType.DMA,
        pltpu.SemaphoreType.DMA,
        pltpu.SemaphoreType.DMA,
        pltpu.SemaphoreType.DMA,
    ],
)
def _sc_encode(tok_hbm, table_hbm, out_hbm, tok_v, rows0, rows1, rows2, rows3,
               out0, out1, sem0, sem1, sem2, sem3, osem0, osem1):
  wid = lax.axis_index("s") * NUM_CORES + lax.axis_index("c")
  sent_base = wid * SENT_PER_TILE

  pltpu.sync_copy(tok_hbm.at[pl.ds(wid * 2 * SENT_PER_TILE, 2 * SENT_PER_TILE)],
                  tok_v)

  NBUF = 4
  LOOKAHEAD = NBUF - 1
  bufs = (rows0, rows1, rows2, rows3)
  sems = (sem0, sem1, sem2, sem3)
  outs = (out0, out1)
  osems = (osem0, osem1)

  for s0 in range(LOOKAHEAD):
    for c in _gather_sent(table_hbm, tok_v, bufs[s0], sems[s0], s0):
      c.start()

  def step(s4, carry):
    for r in range(NBUF):
      s = NBUF * s4 + r
      ob = r % 2
      for c in _gather_sent(table_hbm, tok_v, bufs[r], sems[r], s):
        c.wait()
      @pl.when(s + LOOKAHEAD < SENT_PER_TILE)
      def _():
        nr = (r + LOOKAHEAD) % NBUF
        for c in _gather_sent(table_hbm, tok_v, bufs[nr], sems[nr],
                              s + LOOKAHEAD):
          c.start()
      @pl.when(s >= 2)
      def _():
        pltpu.make_async_copy(
            outs[ob], out_hbm.at[sent_base + s - 2], osems[ob]).wait()
      _pool(bufs[r], outs[ob])
      pltpu.make_async_copy(outs[ob], out_hbm.at[sent_base + s],
                            osems[ob]).start()
    return carry

  lax.fori_loop(0, SENT_PER_TILE // NBUF, step, 0)
  for b in range(2):
    pltpu.make_async_copy(
        outs[b], out_hbm.at[sent_base + SENT_PER_TILE - 2 + b],
        osems[b]).wait()


def kernel(tokens, table):
  tok2 = tokens.astype(jnp.int32).reshape(2 * BATCH, HALF)
  return _sc_encode(tok2, table)
